# Initial kernel scaffold; baseline (speedup 1.0000x reference)
import jax
import jax.numpy as jnp
from jax.experimental import pallas as pl


def kernel(x, adj_vals, W1, b1, W2, b2, edge_index):
    raise NotImplementedError("write your pallas kernel here")


# speedup vs baseline: 1.0232x; 1.0232x over previous
import functools

import jax
import jax.numpy as jnp
from jax.experimental import pallas as pl
from jax.experimental.pallas import tpu as pltpu

_TOPK = 16
_FPAD = 128


def _block_rows(n):
    for br in (400, 256, 250, 200, 128, 125, 100, 64, 50, 32, 25, 16, 8, 5, 4, 2, 1):
        if n % br == 0:
            return br
    return 1


def _weight_xpad_kernel(x_ref, w1_ref, b1_ref, w2_ref, b2_ref, w_ref, xpad_ref):
    x = x_ref[...]
    br, c = x.shape
    m = jnp.max(x, axis=1, keepdims=True)
    z = jnp.sum(jnp.exp(x - m), axis=1, keepdims=True)
    cols = jax.lax.broadcasted_iota(jnp.int32, (br, c), 1)
    work = x
    tvals = []
    for _ in range(_TOPK):
        mk = jnp.max(work, axis=1, keepdims=True)
        fi = jnp.min(jnp.where(work == mk, cols, c + 1), axis=1, keepdims=True)
        tvals.append(mk)
        work = jnp.where(cols == fi, -jnp.inf, work)
    t = jnp.concatenate(tvals, axis=1)
    t = jnp.exp(t - m) / z
    h = jnp.dot(t, w1_ref[...], preferred_element_type=jnp.float32) + b1_ref[...]
    h = jnp.where(h >= 0, h, 0.1 * h)
    zz = jnp.dot(h, w2_ref[...], preferred_element_type=jnp.float32) + b2_ref[...]
    mm = jnp.max(zz, axis=1, keepdims=True)
    e = jnp.exp(zz - mm)
    w_ref[...] = e / jnp.sum(e, axis=1, keepdims=True)
    xpad_ref[...] = jnp.concatenate(
        [x, jnp.zeros((br, _FPAD - c), jnp.float32)], axis=1)


def _combine_kernel(x_ref, pres_ref, w_ref, out_ref):
    x = x_ref[...]
    w = w_ref[...]
    c = x.shape[1]
    d = w.shape[1]
    acc = w[:, 0:1] * x
    for k in range(d - 1):
        acc = acc + w[:, k + 1:k + 2] * pres_ref[k][:, :c]
    m = jnp.max(acc, axis=1, keepdims=True)
    s = acc - m
    out_ref[...] = s - jnp.log(jnp.sum(jnp.exp(s), axis=1, keepdims=True))


def kernel(x, adj_vals, W1, b1, W2, b2, edge_index):
    n, c = x.shape
    degree = W2.shape[1]
    br = _block_rows(n)
    grid = n // br

    weight, xpad = pl.pallas_call(
        _weight_xpad_kernel,
        grid=(grid,),
        in_specs=[
            pl.BlockSpec((br, c), lambda i: (i, 0)),
            pl.BlockSpec(W1.shape, lambda i: (0, 0)),
            pl.BlockSpec((1, W1.shape[1]), lambda i: (0, 0)),
            pl.BlockSpec(W2.shape, lambda i: (0, 0)),
            pl.BlockSpec((1, W2.shape[1]), lambda i: (0, 0)),
        ],
        out_specs=[
            pl.BlockSpec((br, degree), lambda i: (i, 0)),
            pl.BlockSpec((br, _FPAD), lambda i: (i, 0)),
        ],
        out_shape=[
            jax.ShapeDtypeStruct((n, degree), jnp.float32),
            jax.ShapeDtypeStruct((n, _FPAD), jnp.float32),
        ],
    )(x, W1, b1.reshape(1, -1), W2, b2.reshape(1, -1))

    dst = edge_index[0]
    src = edge_index[1]
    pre = xpad
    pres = []
    for _ in range(degree - 1):
        msg = adj_vals[:, None] * pre[src]
        pre = jax.ops.segment_sum(msg, dst, num_segments=n)
        pres.append(pre)
    pres = jnp.stack(pres, axis=0)

    out = pl.pallas_call(
        _combine_kernel,
        grid=(grid,),
        in_specs=[
            pl.BlockSpec((br, c), lambda i: (i, 0)),
            pl.BlockSpec((degree - 1, br, _FPAD), lambda i: (0, i, 0)),
            pl.BlockSpec((br, degree), lambda i: (i, 0)),
        ],
        out_specs=pl.BlockSpec((br, c), lambda i: (i, 0)),
        out_shape=jax.ShapeDtypeStruct((n, c), jnp.float32),
    )(x, pres, weight)
    return out


# speedup vs baseline: 1.7440x; 1.7045x over previous
import functools

import jax
import jax.numpy as jnp
from jax import lax
from jax.experimental import pallas as pl
from jax.experimental.pallas import tpu as pltpu
from jax.experimental.pallas import tpu_sc as plsc

_TOPK = 16
_NC, _NS = 2, 16
_NTILES = _NC * _NS
_GC = 128
_FLUSH = 2048
_HALF = 64


def _block_rows(n):
    for br in (400, 256, 250, 200, 128, 125, 100, 64, 50, 32, 25, 16, 8, 5, 4, 2, 1):
        if n % br == 0:
            return br
    return 1


def _scan_chunk(e):
    for c in (2000, 1600, 1280, 1024, 1000, 800, 640, 512, 400, 320, 256, 160, 128, 80, 64, 32, 16):
        if e % c == 0:
            return c
    return 16


def _weight_xpad_kernel(x_ref, w1_ref, b1_ref, w2_ref, b2_ref, w_ref, xpad_ref):
    x = x_ref[...]
    br, c = x.shape
    m = jnp.max(x, axis=1, keepdims=True)
    z = jnp.sum(jnp.exp(x - m), axis=1, keepdims=True)
    cols = jax.lax.broadcasted_iota(jnp.int32, (br, c), 1)
    work = x
    tvals = []
    for _ in range(_TOPK):
        mk = jnp.max(work, axis=1, keepdims=True)
        fi = jnp.min(jnp.where(work == mk, cols, c + 1), axis=1, keepdims=True)
        tvals.append(mk)
        work = jnp.where(cols == fi, -jnp.inf, work)
    t = jnp.concatenate(tvals, axis=1)
    t = jnp.exp(t - m) / z
    h = jnp.dot(t, w1_ref[...], preferred_element_type=jnp.float32) + b1_ref[...]
    h = jnp.where(h >= 0, h, 0.1 * h)
    zz = jnp.dot(h, w2_ref[...], preferred_element_type=jnp.float32) + b2_ref[...]
    mm = jnp.max(zz, axis=1, keepdims=True)
    e = jnp.exp(zz - mm)
    w_ref[...] = e / jnp.sum(e, axis=1, keepdims=True)
    half = c // 2
    pad = jnp.zeros((br, _HALF - half), jnp.float32)
    xpad_ref[...] = jnp.stack(
        [jnp.concatenate([x[:, :half], pad], axis=1),
         jnp.concatenate([x[:, half:], pad], axis=1)], axis=0)


def _combine_kernel(x_ref, p1, p2, p3, p4, p5, w_ref, out_ref):
    x = x_ref[...]
    w = w_ref[...]
    c = x.shape[1]
    half = c // 2
    acc = w[:, 0:1] * x
    for k, p in enumerate((p1, p2, p3, p4, p5)):
        pk = jnp.concatenate([p[0][:, :half], p[1][:, :half]], axis=1)
        acc = acc + w[:, k + 1:k + 2] * pk
    m = jnp.max(acc, axis=1, keepdims=True)
    s = acc - m
    out_ref[...] = s - jnp.log(jnp.sum(jnp.exp(s), axis=1, keepdims=True))


def _bin_body(dst_hbm, src_hbm, val_hbm, bsrc, bdl, bval, counts,
              dbuf, sbuf, vbuf, sgs, sgd, sgv, cbuf, *, e_total, ch, rn):
    w = lax.axis_index("s") * _NC + lax.axis_index("c")
    lo = w * rn
    hi = lo + rn
    nch = e_total // ch
    nv = ch // 16

    def chunk_body(i, carry):
        cur, gcur = carry
        base = pl.multiple_of(i * ch, 8)
        pltpu.sync_copy(dst_hbm.at[pl.ds(base, ch)], dbuf)
        pltpu.sync_copy(src_hbm.at[pl.ds(base, ch)], sbuf)
        pltpu.sync_copy(val_hbm.at[pl.ds(base, ch)], vbuf)

        def vbody(j, cur):
            d = dbuf[pl.ds(j * 16, 16)]
            m = (d >= lo) & (d < hi)
            plsc.store_compressed(sgs.at[pl.ds(cur, 16)],
                                  sbuf[pl.ds(j * 16, 16)], mask=m)
            plsc.store_compressed(sgd.at[pl.ds(cur, 16)], d - lo, mask=m)
            plsc.store_compressed(sgv.at[pl.ds(cur, 16)],
                                  vbuf[pl.ds(j * 16, 16)], mask=m)
            return cur + plsc.all_reduce_population_count(m)[0]

        cur = lax.fori_loop(0, nv, vbody, cur)

        def do_flush(args):
            cur, gcur = args
            gcur = pl.multiple_of(gcur, 8)
            pltpu.sync_copy(sgs.at[pl.ds(0, _FLUSH)],
                            bsrc.at[w, pl.ds(gcur, _FLUSH)])
            pltpu.sync_copy(sgd.at[pl.ds(0, _FLUSH)],
                            bdl.at[w, pl.ds(gcur, _FLUSH)])
            pltpu.sync_copy(sgv.at[pl.ds(0, _FLUSH)],
                            bval.at[w, pl.ds(gcur, _FLUSH)])

            def mv(j, _):
                sgs[pl.ds(j * 16, 16)] = sgs[pl.ds(_FLUSH + j * 16, 16)]
                sgd[pl.ds(j * 16, 16)] = sgd[pl.ds(_FLUSH + j * 16, 16)]
                sgv[pl.ds(j * 16, 16)] = sgv[pl.ds(_FLUSH + j * 16, 16)]
                return 0

            lax.fori_loop(0, ch // 16, mv, 0)
            return cur - _FLUSH, gcur + _FLUSH

        return lax.cond(cur >= _FLUSH, do_flush, lambda a: a, (cur, gcur))

    cur, gcur = lax.fori_loop(
        0, nch, chunk_body, (jnp.int32(0), jnp.int32(0)))

    zi = jnp.zeros((16,), jnp.int32)
    zf = jnp.zeros((16,), jnp.float32)
    di = jnp.full((16,), rn, jnp.int32)
    for j in range(_GC // 16):
        sgs[pl.ds(cur + j * 16, 16)] = zi
        sgd[pl.ds(cur + j * 16, 16)] = di
        sgv[pl.ds(cur + j * 16, 16)] = zf
    cur = ((cur + _GC - 1) // _GC) * _GC

    def final_flush(args):
        cur, gcur = args
        gcur = pl.multiple_of(gcur, 8)
        pltpu.sync_copy(sgs.at[pl.ds(0, _FLUSH)],
                        bsrc.at[w, pl.ds(gcur, _FLUSH)])
        pltpu.sync_copy(sgd.at[pl.ds(0, _FLUSH)],
                        bdl.at[w, pl.ds(gcur, _FLUSH)])
        pltpu.sync_copy(sgv.at[pl.ds(0, _FLUSH)],
                        bval.at[w, pl.ds(gcur, _FLUSH)])
        return args

    lax.cond(cur > 0, final_flush, lambda a: a, (cur, gcur))
    cbuf[pl.ds(0, 16)] = jnp.full((16,), gcur + cur, jnp.int32)
    pltpu.sync_copy(cbuf.at[pl.ds(0, 8)], counts.at[w])


def _hop_body(pre, bsrc, bdl, bval, counts, out,
              sbuf, dbuf, vbuf, rows, acc, cnt8, sem, *, rn, racc):
    w = lax.axis_index("s") * _NC + lax.axis_index("c")
    lo = w * rn
    pltpu.sync_copy(counts.at[w], cnt8.at[pl.ds(0, 8)])
    cnt = cnt8[pl.ds(0, 16)][0]
    nch = cnt // _GC
    zf = jnp.zeros((16,), jnp.float32)
    for h in range(2):
        def zero_body(r, _):
            for j in range(4):
                acc[r, pl.ds(j * 16, 16)] = zf
            return 0
        lax.fori_loop(0, racc, zero_body, 0)

        def chunk_body(i, _):
            base = pl.multiple_of(i * _GC, 8)
            pltpu.sync_copy(bsrc.at[w, pl.ds(base, _GC)], sbuf)
            pltpu.sync_copy(bdl.at[w, pl.ds(base, _GC)], dbuf)
            pltpu.sync_copy(bval.at[w, pl.ds(base, _GC)], vbuf)
            pltpu.async_copy(pre.at[h].at[sbuf], rows, sem).wait()

            def fma_body(k, _):
                dlv = dbuf[pl.ds(k * 16, 16)]
                vv = vbuf[pl.ds(k * 16, 16)]
                for u in range(16):
                    e = k * 16 + u
                    dl = dlv[u]
                    v = vv[u]
                    for j in range(4):
                        plsc.addupdate(acc.at[dl, pl.ds(j * 16, 16)],
                                       v * rows[e, pl.ds(j * 16, 16)])
                return 0

            lax.fori_loop(0, _GC // 16, fma_body, 0)
            return 0

        lax.fori_loop(0, nch, chunk_body, 0)
        pltpu.sync_copy(acc.at[pl.ds(0, rn)], out.at[h].at[pl.ds(lo, rn)])


def kernel(x, adj_vals, W1, b1, W2, b2, edge_index):
    n, c = x.shape
    e_total = adj_vals.shape[0]
    degree = W2.shape[1]
    br = _block_rows(n)
    grid = n // br
    rn = -(-n // _NTILES)
    npad = _NTILES * rn
    racc = ((rn + 1 + 7) // 8) * 8
    eb = e_total + 2 * _FLUSH
    ch = _scan_chunk(e_total)

    weight, xpad = pl.pallas_call(
        _weight_xpad_kernel,
        grid=(grid,),
        in_specs=[
            pl.BlockSpec((br, c), lambda i: (i, 0)),
            pl.BlockSpec(W1.shape, lambda i: (0, 0)),
            pl.BlockSpec((1, W1.shape[1]), lambda i: (0, 0)),
            pl.BlockSpec(W2.shape, lambda i: (0, 0)),
            pl.BlockSpec((1, W2.shape[1]), lambda i: (0, 0)),
        ],
        out_specs=[
            pl.BlockSpec((br, degree), lambda i: (i, 0)),
            pl.BlockSpec((2, br, _HALF), lambda i: (0, i, 0)),
        ],
        out_shape=[
            jax.ShapeDtypeStruct((n, degree), jnp.float32),
            jax.ShapeDtypeStruct((2, npad, _HALF), jnp.float32),
        ],
    )(x, W1, b1.reshape(1, -1), W2, b2.reshape(1, -1))

    mesh = plsc.VectorSubcoreMesh(core_axis_name="c", subcore_axis_name="s",
                                  num_cores=_NC, num_subcores=_NS)
    sc_params = pltpu.CompilerParams(use_tc_tiling_on_sc=False,
                                     needs_layout_passes=False)
    i32 = jnp.int32
    bin_fn = pl.kernel(
        functools.partial(_bin_body, e_total=e_total, ch=ch, rn=rn),
        out_type=[
            jax.ShapeDtypeStruct((_NTILES, eb), i32),
            jax.ShapeDtypeStruct((_NTILES, eb), i32),
            jax.ShapeDtypeStruct((_NTILES, eb), jnp.float32),
            jax.ShapeDtypeStruct((_NTILES, 8), i32),
        ],
        mesh=mesh,
        scratch_types=[
            pltpu.VMEM((ch,), i32),
            pltpu.VMEM((ch,), i32),
            pltpu.VMEM((ch,), jnp.float32),
            pltpu.VMEM((_FLUSH + ch + _GC + 16,), i32),
            pltpu.VMEM((_FLUSH + ch + _GC + 16,), i32),
            pltpu.VMEM((_FLUSH + ch + _GC + 16,), jnp.float32),
            pltpu.VMEM((16,), i32),
        ],
        compiler_params=sc_params,
    )
    bsrc, bdl, bval, counts = bin_fn(
        edge_index[0], edge_index[1], adj_vals)

    hop_fn = pl.kernel(
        functools.partial(_hop_body, rn=rn, racc=racc),
        out_type=jax.ShapeDtypeStruct((2, npad, _HALF), jnp.float32),
        mesh=mesh,
        scratch_types=[
            pltpu.VMEM((_GC,), i32),
            pltpu.VMEM((_GC,), i32),
            pltpu.VMEM((_GC,), jnp.float32),
            pltpu.VMEM((_GC, _HALF), jnp.float32),
            pltpu.VMEM((racc, _HALF), jnp.float32),
            pltpu.VMEM((16,), i32),
            pltpu.SemaphoreType.DMA,
        ],
        compiler_params=sc_params,
    )

    pres = []
    pre = xpad
    for _ in range(degree - 1):
        pre = hop_fn(pre, bsrc, bdl, bval, counts)
        pres.append(pre)

    out = pl.pallas_call(
        _combine_kernel,
        grid=(grid,),
        in_specs=[pl.BlockSpec((br, c), lambda i: (i, 0))]
        + [pl.BlockSpec((2, br, _HALF), lambda i: (0, i, 0))] * (degree - 1)
        + [pl.BlockSpec((br, degree), lambda i: (i, 0))],
        out_specs=pl.BlockSpec((br, c), lambda i: (i, 0)),
        out_shape=jax.ShapeDtypeStruct((n, c), jnp.float32),
    )(x, *pres, weight)
    return out
